# baseline (speedup 1.0000x reference)
import functools

import jax
import jax.numpy as jnp
from jax import lax
from jax.experimental import pallas as pl
from jax.experimental.pallas import tpu as pltpu
from jax.experimental.pallas import tpu_sc as plsc

_BM = 2000
_CH = 80


def _relu(v):
    return jnp.maximum(v, 0.0)


def _fnn2(h, w1, b1, w2, b2):
    return _relu(h @ w1[...] + b1[...]) @ w2[...] + b2[...]


def _head_body(x_ref, nw1, nb1, nw2, nb2, fw1, fb1, fw2, fb2,
               bw1, bb1, bw2, bb2, st_ref, tf_ref, tb_ref):
    st = _fnn2(x_ref[...], nw1, nb1, nw2, nb2)
    st_ref[...] = st
    tf_ref[...] = _relu(_fnn2(st, fw1, fb1, fw2, fb2))
    tb_ref[...] = _relu(_fnn2(st, bw1, bb1, bw2, bb2))


def _combine_mid_body(n_total, zf_ref, zb_ref, st_ref,
                      ufw1, ufb1, ufw2, ufb2, pfw1, pfb1, pfw2, pfb2,
                      ubw1, ubb1, ubw2, ubb2, pbw1, pbb1, pbw2, pbb2,
                      tf_ref, tb_ref):
    i = pl.program_id(0)
    st = st_ref[...]
    rows = i * _BM + lax.broadcasted_iota(jnp.int32, (_BM, 1), 0)
    hf = _relu(_fnn2(zf_ref[0], ufw1, ufb1, ufw2, ufb2))
    hf = jnp.where(rows == n_total - 1, 0.0, hf)
    tf_ref[...] = _relu(_fnn2(st + hf, pfw1, pfb1, pfw2, pfb2))
    hb = _relu(_fnn2(zb_ref[0], ubw1, ubb1, ubw2, ubb2))
    hb = jnp.where(rows == 0, 0.0, hb)
    tb_ref[...] = _relu(_fnn2(st + hb, pbw1, pbb1, pbw2, pbb2))


def _combine_final_body(n_total, d, zf_ref, zb_ref, st_ref,
                        ufw1, ufb1, ufw2, ufb2, ubw1, ubb1, ubw2, ubb2,
                        out_ref):
    i = pl.program_id(0)
    st = st_ref[...]
    rows = i * _BM + lax.broadcasted_iota(jnp.int32, (_BM, 1), 0)
    hf = _relu(_fnn2(zf_ref[0], ufw1, ufb1, ufw2, ufb2))
    hf = jnp.where(rows == n_total - 1, 0.0, hf)
    out_ref[:, :d] = st + hf
    hb = _relu(_fnn2(zb_ref[0], ubw1, ubb1, ubw2, ubb2))
    hb = jnp.where(rows == 0, 0.0, hb)
    out_ref[:, d:] = st + hb


_G = 5


def _make_segsum(n, d, e):
    info = plsc.get_sparse_core_info()
    ns = info.num_subcores
    cpw = (e // _CH) // ns
    rpw = n // ns
    ng = cpw // _G
    mesh = plsc.VectorSubcoreMesh(core_axis_name="c", subcore_axis_name="s")

    def body(tf_hbm, tb_hbm, er_hbm, zeros_hbm, z2_hbm,
             gib, sib, rows_v, zsh, sga, sgb, sia, sibm, sz):
        cid = lax.axis_index("c")
        sid = lax.axis_index("s")
        pltpu.async_copy(zeros_hbm, zsh.at[pl.ds(sid * rpw, rpw)], sz)

        def direction(t_src, g_row, s_row):
            sg = (sga, sgb)
            si_sem = (sia, sibm)

            def start_gather(p, j, r):
                pltpu.async_copy(t_src.at[gib.at[p, j]], rows_v.at[r], sg[r])

            def wait_gather(p, j, r):
                pltpu.make_async_copy(
                    t_src.at[gib.at[p, j]], rows_v.at[r], sg[r]).wait()

            def scatter(p, j, r):
                pltpu.sync_copy(rows_v.at[r], zsh.at[sib.at[p, j]], add=True)

            def load_idx(g, p):
                pltpu.async_copy(er_hbm.at[g_row, sid, g], gib.at[p],
                                 si_sem[p])
                pltpu.async_copy(er_hbm.at[s_row, sid, g], sib.at[p],
                                 si_sem[p])

            def wait_idx(p):
                pltpu.make_async_copy(
                    er_hbm.at[g_row, sid, 0], gib.at[p], si_sem[p]).wait()
                pltpu.make_async_copy(
                    er_hbm.at[s_row, sid, 0], sib.at[p], si_sem[p]).wait()

            pltpu.sync_copy(er_hbm.at[g_row, sid, 0], gib.at[0])
            pltpu.sync_copy(er_hbm.at[s_row, sid, 0], sib.at[0])
            load_idx(1, 1)
            start_gather(0, 0, 0)
            pltpu.make_async_copy(
                zeros_hbm, zsh.at[pl.ds(sid * rpw, rpw)], sz).wait()
            plsc.subcore_barrier()

            def pair(m, carry):
                for j in range(1, _G):
                    start_gather(0, j, j % 2)
                    wait_gather(0, j - 1, (j + 1) % 2)
                    scatter(0, j - 1, (j + 1) % 2)
                wait_idx(1)
                start_gather(1, 0, 1)
                wait_gather(0, _G - 1, 0)
                scatter(0, _G - 1, 0)
                load_idx(jnp.minimum(2 * m + 2, ng - 1), 0)
                for j in range(1, _G):
                    start_gather(1, j, (_G + j) % 2)
                    wait_gather(1, j - 1, (_G + j + 1) % 2)
                    scatter(1, j - 1, (_G + j + 1) % 2)
                wait_idx(0)
                start_gather(0, 0, 0)
                wait_gather(1, _G - 1, 1)
                scatter(1, _G - 1, 1)
                load_idx(jnp.minimum(2 * m + 3, ng - 1), 1)
                return carry

            lax.fori_loop(0, ng // 2, pair, 0)
            wait_gather(0, 0, 0)
            wait_idx(1)

        @pl.when(cid == 0)
        def _():
            direction(tf_hbm, 0, 1)

        @pl.when(cid == 1)
        def _():
            direction(tb_hbm, 1, 0)

        plsc.subcore_barrier()

        @pl.when(cid == 0)
        def _():
            pltpu.sync_copy(zsh.at[pl.ds(sid * rpw, rpw)], z2_hbm.at[0, sid])

        @pl.when(cid == 1)
        def _():
            pltpu.sync_copy(zsh.at[pl.ds(sid * rpw, rpw)], z2_hbm.at[1, sid])

    return pl.kernel(
        body,
        out_type=jax.ShapeDtypeStruct((2, ns, rpw, d), jnp.float32),
        mesh=mesh,
        scratch_types=[
            pltpu.VMEM((2, _G, _CH), jnp.int32),
            pltpu.VMEM((2, _G, _CH), jnp.int32),
            pltpu.VMEM((2, _CH, d), jnp.float32),
            pltpu.VMEM_SHARED((n, d), jnp.float32),
            pltpu.SemaphoreType.DMA,
            pltpu.SemaphoreType.DMA,
            pltpu.SemaphoreType.DMA,
            pltpu.SemaphoreType.DMA,
            pltpu.SemaphoreType.DMA,
        ],
    )


def kernel(x, edge_index, params):
    n, d = x.shape
    e = edge_index.shape[1]
    grid = n // _BM

    def wb(p):
        return (p['W1'], p['b1'].reshape(1, -1), p['W2'], p['b2'].reshape(1, -1))

    wspec = [pl.BlockSpec((d, d), lambda i: (0, 0)),
             pl.BlockSpec((1, d), lambda i: (0, 0)),
             pl.BlockSpec((d, d), lambda i: (0, 0)),
             pl.BlockSpec((1, d), lambda i: (0, 0))]
    row = pl.BlockSpec((_BM, d), lambda i: (i, 0))
    zf_spec = pl.BlockSpec((1, _BM, d), lambda i: (0, i, 0))
    zb_spec = pl.BlockSpec((1, _BM, d), lambda i: (1, i, 0))
    nd = jax.ShapeDtypeStruct((n, d), jnp.float32)

    head = pl.pallas_call(
        _head_body,
        grid=(grid,),
        in_specs=[row] + wspec * 3,
        out_specs=[row, row, row],
        out_shape=[nd, nd, nd],
    )
    st, tf, tb = head(x, *wb(params['nt']), *wb(params['fpa_pre']),
                      *wb(params['bpa_pre']))

    combine_mid = pl.pallas_call(
        functools.partial(_combine_mid_body, n),
        grid=(grid,),
        in_specs=[zf_spec, zb_spec, row] + wspec * 4,
        out_specs=[row, row],
        out_shape=[nd, nd],
    )
    combine_final = pl.pallas_call(
        functools.partial(_combine_final_body, n, d),
        grid=(grid,),
        in_specs=[zf_spec, zb_spec, row] + wspec * 2,
        out_specs=pl.BlockSpec((_BM, 2 * d), lambda i: (i, 0)),
        out_shape=jax.ShapeDtypeStruct((n, 2 * d), jnp.float32),
    )

    segsum = _make_segsum(n, d, e)
    ns = plsc.get_sparse_core_info().num_subcores
    ng = (e // _CH) // ns // _G
    er = edge_index.reshape(2, ns, ng, _G, _CH)
    zeros = jnp.zeros((n // ns, d), jnp.float32)

    for _ in range(2):
        z2 = segsum(tf, tb, er, zeros).reshape(2, n, d)
        tf, tb = combine_mid(z2, z2, st, *wb(params['fpa_upd']),
                             *wb(params['fpa_pre']), *wb(params['bpa_upd']),
                             *wb(params['bpa_pre']))
    z2 = segsum(tf, tb, er, zeros).reshape(2, n, d)
    return combine_final(z2, z2, st, *wb(params['fpa_upd']),
                         *wb(params['bpa_upd']))


# speedup vs baseline: 1.0399x; 1.0399x over previous
import functools

import jax
import jax.numpy as jnp
from jax import lax
from jax.experimental import pallas as pl
from jax.experimental.pallas import tpu as pltpu
from jax.experimental.pallas import tpu_sc as plsc

_BM = 2000
_CH = 80


def _relu(v):
    return jnp.maximum(v, 0.0)


def _fnn2(h, w1, b1, w2, b2):
    return _relu(h @ w1[...] + b1[...]) @ w2[...] + b2[...]


def _head_body(x_ref, nw1, nb1, nw2, nb2, fw1, fb1, fw2, fb2,
               bw1, bb1, bw2, bb2, st_ref, tf_ref, tb_ref):
    st = _fnn2(x_ref[...], nw1, nb1, nw2, nb2)
    st_ref[...] = st
    tf_ref[...] = _relu(_fnn2(st, fw1, fb1, fw2, fb2))
    tb_ref[...] = _relu(_fnn2(st, bw1, bb1, bw2, bb2))


def _combine_mid_body(n_total, zf_ref, zb_ref, st_ref,
                      ufw1, ufb1, ufw2, ufb2, pfw1, pfb1, pfw2, pfb2,
                      ubw1, ubb1, ubw2, ubb2, pbw1, pbb1, pbw2, pbb2,
                      tf_ref, tb_ref):
    i = pl.program_id(0)
    st = st_ref[...]
    rows = i * _BM + lax.broadcasted_iota(jnp.int32, (_BM, 1), 0)
    hf = _relu(_fnn2(zf_ref[0], ufw1, ufb1, ufw2, ufb2))
    hf = jnp.where(rows == n_total - 1, 0.0, hf)
    tf_ref[...] = _relu(_fnn2(st + hf, pfw1, pfb1, pfw2, pfb2))
    hb = _relu(_fnn2(zb_ref[0], ubw1, ubb1, ubw2, ubb2))
    hb = jnp.where(rows == 0, 0.0, hb)
    tb_ref[...] = _relu(_fnn2(st + hb, pbw1, pbb1, pbw2, pbb2))


def _combine_final_body(n_total, d, zf_ref, zb_ref, st_ref,
                        ufw1, ufb1, ufw2, ufb2, ubw1, ubb1, ubw2, ubb2,
                        out_ref):
    i = pl.program_id(0)
    st = st_ref[...]
    rows = i * _BM + lax.broadcasted_iota(jnp.int32, (_BM, 1), 0)
    hf = _relu(_fnn2(zf_ref[0], ufw1, ufb1, ufw2, ufb2))
    hf = jnp.where(rows == n_total - 1, 0.0, hf)
    out_ref[:, :d] = st + hf
    hb = _relu(_fnn2(zb_ref[0], ubw1, ubb1, ubw2, ubb2))
    hb = jnp.where(rows == 0, 0.0, hb)
    out_ref[:, d:] = st + hb


_G = 5


def _make_segsum(n, d, e):
    info = plsc.get_sparse_core_info()
    ns = info.num_subcores
    cpw = (e // _CH) // ns
    rs = 8 * ((n // ns) // 8)
    rlast = n - rs * (ns - 1)
    ng = cpw // _G
    mesh = plsc.VectorSubcoreMesh(core_axis_name="c", subcore_axis_name="s")

    def body(tf_hbm, tb_hbm, er_hbm, zeros_hbm, z2_hbm,
             gib, sib, rows_v, zsh, sga, sgb, sia, sibm, sz):
        cid = lax.axis_index("c")
        sid = lax.axis_index("s")
        @pl.when(sid < ns - 1)
        def _():
            pltpu.async_copy(zeros_hbm.at[pl.ds(0, rs)],
                             zsh.at[pl.ds(sid * rs, rs)], sz)

        @pl.when(sid == ns - 1)
        def _():
            pltpu.async_copy(zeros_hbm, zsh.at[pl.ds((ns - 1) * rs, rlast)],
                             sz)

        def zinit_wait():
            @pl.when(sid < ns - 1)
            def _():
                pltpu.make_async_copy(zeros_hbm.at[pl.ds(0, rs)],
                                      zsh.at[pl.ds(sid * rs, rs)], sz).wait()

            @pl.when(sid == ns - 1)
            def _():
                pltpu.make_async_copy(
                    zeros_hbm,
                    zsh.at[pl.ds((ns - 1) * rs, rlast)], sz).wait()

        def direction(t_src, g_row, s_row):
            sg = (sga, sgb)
            si_sem = (sia, sibm)

            def start_gather(p, j, r):
                pltpu.async_copy(t_src.at[gib.at[p, j]], rows_v.at[r], sg[r])

            def wait_gather(p, j, r):
                pltpu.make_async_copy(
                    t_src.at[gib.at[p, j]], rows_v.at[r], sg[r]).wait()

            def scatter(p, j, r):
                pltpu.sync_copy(rows_v.at[r], zsh.at[sib.at[p, j]], add=True)

            def load_idx(g, p):
                pltpu.async_copy(er_hbm.at[g_row, sid, g], gib.at[p],
                                 si_sem[p])
                pltpu.async_copy(er_hbm.at[s_row, sid, g], sib.at[p],
                                 si_sem[p])

            def wait_idx(p):
                pltpu.make_async_copy(
                    er_hbm.at[g_row, sid, 0], gib.at[p], si_sem[p]).wait()
                pltpu.make_async_copy(
                    er_hbm.at[s_row, sid, 0], sib.at[p], si_sem[p]).wait()

            pltpu.sync_copy(er_hbm.at[g_row, sid, 0], gib.at[0])
            pltpu.sync_copy(er_hbm.at[s_row, sid, 0], sib.at[0])
            load_idx(1, 1)
            start_gather(0, 0, 0)
            zinit_wait()
            plsc.subcore_barrier()

            def pair(m, carry):
                for j in range(1, _G):
                    start_gather(0, j, j % 2)
                    wait_gather(0, j - 1, (j + 1) % 2)
                    scatter(0, j - 1, (j + 1) % 2)
                wait_idx(1)
                start_gather(1, 0, 1)
                wait_gather(0, _G - 1, 0)
                scatter(0, _G - 1, 0)
                load_idx(jnp.minimum(2 * m + 2, ng - 1), 0)
                for j in range(1, _G):
                    start_gather(1, j, (_G + j) % 2)
                    wait_gather(1, j - 1, (_G + j + 1) % 2)
                    scatter(1, j - 1, (_G + j + 1) % 2)
                wait_idx(0)
                start_gather(0, 0, 0)
                wait_gather(1, _G - 1, 1)
                scatter(1, _G - 1, 1)
                load_idx(jnp.minimum(2 * m + 3, ng - 1), 1)
                return carry

            lax.fori_loop(0, ng // 2, pair, 0)
            wait_gather(0, 0, 0)
            wait_idx(1)

        @pl.when(cid == 0)
        def _():
            direction(tf_hbm, 0, 1)

        @pl.when(cid == 1)
        def _():
            direction(tb_hbm, 1, 0)

        plsc.subcore_barrier()

        def readback(dirc):
            @pl.when(sid < ns - 1)
            def _():
                pltpu.sync_copy(zsh.at[pl.ds(sid * rs, rs)],
                                z2_hbm.at[dirc, pl.ds(sid * rs, rs)])

            @pl.when(sid == ns - 1)
            def _():
                pltpu.sync_copy(
                    zsh.at[pl.ds((ns - 1) * rs, rlast)],
                    z2_hbm.at[dirc, pl.ds((ns - 1) * rs, rlast)])

        @pl.when(cid == 0)
        def _():
            readback(0)

        @pl.when(cid == 1)
        def _():
            readback(1)

    return pl.kernel(
        body,
        out_type=jax.ShapeDtypeStruct((2, n, d), jnp.float32),
        mesh=mesh,
        scratch_types=[
            pltpu.VMEM((2, _G, _CH), jnp.int32),
            pltpu.VMEM((2, _G, _CH), jnp.int32),
            pltpu.VMEM((2, _CH, d), jnp.float32),
            pltpu.VMEM_SHARED((n, d), jnp.float32),
            pltpu.SemaphoreType.DMA,
            pltpu.SemaphoreType.DMA,
            pltpu.SemaphoreType.DMA,
            pltpu.SemaphoreType.DMA,
            pltpu.SemaphoreType.DMA,
        ],
    )


def kernel(x, edge_index, params):
    n, d = x.shape
    e = edge_index.shape[1]
    grid = n // _BM

    def wb(p):
        return (p['W1'], p['b1'].reshape(1, -1), p['W2'], p['b2'].reshape(1, -1))

    wspec = [pl.BlockSpec((d, d), lambda i: (0, 0)),
             pl.BlockSpec((1, d), lambda i: (0, 0)),
             pl.BlockSpec((d, d), lambda i: (0, 0)),
             pl.BlockSpec((1, d), lambda i: (0, 0))]
    row = pl.BlockSpec((_BM, d), lambda i: (i, 0))
    zf_spec = pl.BlockSpec((1, _BM, d), lambda i: (0, i, 0))
    zb_spec = pl.BlockSpec((1, _BM, d), lambda i: (1, i, 0))
    nd = jax.ShapeDtypeStruct((n, d), jnp.float32)

    head = pl.pallas_call(
        _head_body,
        grid=(grid,),
        in_specs=[row] + wspec * 3,
        out_specs=[row, row, row],
        out_shape=[nd, nd, nd],
    )
    st, tf, tb = head(x, *wb(params['nt']), *wb(params['fpa_pre']),
                      *wb(params['bpa_pre']))

    combine_mid = pl.pallas_call(
        functools.partial(_combine_mid_body, n),
        grid=(grid,),
        in_specs=[zf_spec, zb_spec, row] + wspec * 4,
        out_specs=[row, row],
        out_shape=[nd, nd],
    )
    combine_final = pl.pallas_call(
        functools.partial(_combine_final_body, n, d),
        grid=(grid,),
        in_specs=[zf_spec, zb_spec, row] + wspec * 2,
        out_specs=pl.BlockSpec((_BM, 2 * d), lambda i: (i, 0)),
        out_shape=jax.ShapeDtypeStruct((n, 2 * d), jnp.float32),
    )

    segsum = _make_segsum(n, d, e)
    ns = plsc.get_sparse_core_info().num_subcores
    ng = (e // _CH) // ns // _G
    er = edge_index.reshape(2, ns, ng, _G, _CH)
    rlast = n - 8 * ((n // ns) // 8) * (ns - 1)
    zeros = jnp.zeros((rlast, d), jnp.float32)

    for _ in range(2):
        z2 = segsum(tf, tb, er, zeros)
        tf, tb = combine_mid(z2, z2, st, *wb(params['fpa_upd']),
                             *wb(params['fpa_pre']), *wb(params['bpa_upd']),
                             *wb(params['bpa_pre']))
    z2 = segsum(tf, tb, er, zeros)
    return combine_final(z2, z2, st, *wb(params['fpa_upd']),
                         *wb(params['bpa_upd']))
